# baseline (speedup 1.0000x reference)
import jax
import jax.numpy as jnp
from jax.experimental import pallas as pl
from jax.experimental.pallas import tpu as pltpu

EPS_ = 1e-5
ROWS, COLS = 65536, 512
CH = 2048
NCH = ROWS // CH
RD = 8
WD = 5
MD = 4
NOUTER = NCH // RD


def _bn_kernel(x_hbm, m_hbm, m2_hbm, g_hbm, b_hbm, o_hbm,
               xbuf, mbuf, obuf, gloc, bloc, mloc,
               acc_s, acc_q, coef_c, coef_b,
               sem_rx, sem_rm, sem_w, sem_misc):

    def read_x(j, s):
        return pltpu.make_async_copy(
            x_hbm.at[pl.ds(j * CH, CH), :], xbuf.at[s], sem_rx.at[s])

    def read_m(j, s):
        return pltpu.make_async_copy(
            m_hbm.at[pl.ds(j * CH, CH), :], mbuf.at[s], sem_rm.at[s])

    def write_o(j, s):
        return pltpu.make_async_copy(
            obuf.at[s], o_hbm.at[pl.ds(j * CH, CH), :], sem_w.at[s])

    cg = pltpu.make_async_copy(g_hbm, gloc, sem_misc.at[0])
    cb = pltpu.make_async_copy(b_hbm, bloc, sem_misc.at[1])
    cm = pltpu.make_async_copy(m2_hbm, mloc, sem_misc.at[2])
    cg.start()
    cb.start()
    cm.start()

    acc_s[...] = jnp.zeros_like(acc_s)
    acc_q[...] = jnp.zeros_like(acc_q)

    for s in range(RD):
        read_x(s, s).start()
    for s in range(MD):
        read_m(s, s).start()

    def p0_body(j2, carry):
        for s in range(RD):
            j = j2 * RD + s
            read_x(j, s).wait()
            read_m(j, j % MD).wait()
            x = xbuf[s].reshape(CH // 64, 64, COLS)
            m = mbuf[j % MD].reshape(CH // 64, 64, 1)
            xm = x * m
            acc_s[...] += jnp.sum(xm, axis=0)
            acc_q[...] += jnp.sum(xm * xm, axis=0)

            @pl.when(j + RD < NCH)
            def _():
                read_x(j + RD, s).start()

            @pl.when(j + MD < NCH)
            def _():
                read_m(j + MD, j % MD).start()
        return carry

    jax.lax.fori_loop(0, NOUTER, p0_body, 0)

    cg.wait()
    cb.wait()
    cm.wait()
    cnt_l = jnp.sum(mloc[...], axis=0, keepdims=True)
    cnt = jnp.broadcast_to(jnp.sum(cnt_l, axis=1, keepdims=True), (1, COLS))
    mean = jnp.sum(acc_s[...], axis=0, keepdims=True) / cnt
    var = jnp.sum(acc_q[...], axis=0, keepdims=True) / cnt - mean * mean
    a = jax.lax.rsqrt(var + EPS_) * gloc[...]
    coef_c[...] = a - 1.0
    coef_b[...] = bloc[...] - mean * a

    for s in range(RD):
        read_x(s, s).start()
    for s in range(MD):
        read_m(s, s).start()

    def p1_body(j2, carry):
        for s in range(RD):
            j = j2 * RD + s
            read_x(j, s).wait()
            read_m(j, j % MD).wait()

            @pl.when(j >= WD)
            def _():
                write_o(j - WD, j % WD).wait()

            x = xbuf[s]
            m = mbuf[j % MD]
            t = x * coef_c[...] + coef_b[...]
            obuf[j % WD] = x + t * m
            write_o(j, j % WD).start()

            @pl.when(j + RD < NCH)
            def _():
                read_x(j + RD, s).start()

            @pl.when(j + MD < NCH)
            def _():
                read_m(j + MD, j % MD).start()
        return carry

    jax.lax.fori_loop(0, NOUTER, p1_body, 0)

    for s in range(WD):
        write_o(NCH - WD + s, (NCH - WD + s) % WD).wait()


def kernel(x, mask, gamma, beta):
    m = mask.astype(jnp.float32).reshape(ROWS, 1)
    m2 = mask.astype(jnp.float32).reshape(ROWS // 128, 128)
    g = gamma.reshape(1, COLS)
    b = beta.reshape(1, COLS)
    out = pl.pallas_call(
        _bn_kernel,
        in_specs=[
            pl.BlockSpec(memory_space=pl.ANY),
            pl.BlockSpec(memory_space=pl.ANY),
            pl.BlockSpec(memory_space=pl.ANY),
            pl.BlockSpec(memory_space=pl.ANY),
            pl.BlockSpec(memory_space=pl.ANY),
        ],
        out_specs=pl.BlockSpec(memory_space=pl.ANY),
        out_shape=jax.ShapeDtypeStruct((ROWS, COLS), x.dtype),
        scratch_shapes=[
            pltpu.VMEM((RD, CH, COLS), jnp.float32),
            pltpu.VMEM((MD, CH, 1), jnp.float32),
            pltpu.VMEM((WD, CH, COLS), jnp.float32),
            pltpu.VMEM((1, COLS), jnp.float32),
            pltpu.VMEM((1, COLS), jnp.float32),
            pltpu.VMEM((ROWS // 128, 128), jnp.float32),
            pltpu.VMEM((64, COLS), jnp.float32),
            pltpu.VMEM((64, COLS), jnp.float32),
            pltpu.VMEM((1, COLS), jnp.float32),
            pltpu.VMEM((1, COLS), jnp.float32),
            pltpu.SemaphoreType.DMA((RD,)),
            pltpu.SemaphoreType.DMA((MD,)),
            pltpu.SemaphoreType.DMA((WD,)),
            pltpu.SemaphoreType.DMA((3,)),
        ],
    )(x, m, m2, g, b)
    return out


# speedup vs baseline: 1.0607x; 1.0607x over previous
import jax
import jax.numpy as jnp
from jax.experimental import pallas as pl
from jax.experimental.pallas import tpu as pltpu

EPS_ = 1e-5
ROWS, COLS = 65536, 512
CH = 2048
NCH = ROWS // CH
RD = 8
WD = 3
MD = 4
NOUTER = NCH // RD


def _bn_kernel(x_hbm, m_hbm, m2_hbm, g_hbm, b_hbm, o_hbm,
               xbuf, mbuf, gloc, bloc, mloc,
               acc_s, acc_q, coef_c, coef_b,
               sem_rx, sem_rm, sem_w, sem_misc):

    def read_x(j, s):
        return pltpu.make_async_copy(
            x_hbm.at[pl.ds(j * CH, CH), :], xbuf.at[s], sem_rx.at[s])

    def read_m(j, s):
        return pltpu.make_async_copy(
            m_hbm.at[pl.ds(j * CH, CH), :], mbuf.at[s], sem_rm.at[s])

    def write_o(c):
        return pltpu.make_async_copy(
            xbuf.at[c % RD], o_hbm.at[pl.ds(c * CH, CH), :], sem_w.at[c % RD])

    cg = pltpu.make_async_copy(g_hbm, gloc, sem_misc.at[0])
    cb = pltpu.make_async_copy(b_hbm, bloc, sem_misc.at[1])
    cm = pltpu.make_async_copy(m2_hbm, mloc, sem_misc.at[2])
    cg.start()
    cb.start()
    cm.start()

    acc_s[...] = jnp.zeros_like(acc_s)
    acc_q[...] = jnp.zeros_like(acc_q)

    for s in range(RD):
        read_x(s, s).start()
    for s in range(MD):
        read_m(s, s).start()

    def p0_body(j2, carry):
        for s in range(RD):
            j = j2 * RD + s
            read_x(j, s).wait()
            read_m(j, j % MD).wait()
            x = xbuf[s].reshape(CH // 32, 32, COLS)
            m = mbuf[j % MD].reshape(CH // 32, 32, 1)
            xm = x * m
            acc_s[...] += jnp.sum(xm, axis=0)
            acc_q[...] += jnp.sum(xm * xm, axis=0)

            @pl.when(j + RD < NCH)
            def _():
                read_x(j + RD, s).start()

            @pl.when(j + MD < NCH)
            def _():
                read_m(j + MD, j % MD).start()
        return carry

    jax.lax.fori_loop(0, NOUTER, p0_body, 0)

    cg.wait()
    cb.wait()
    cm.wait()
    cnt_l = jnp.sum(mloc[...], axis=0, keepdims=True)
    cnt = jnp.broadcast_to(jnp.sum(cnt_l, axis=1, keepdims=True), (1, COLS))
    mean = jnp.sum(acc_s[...], axis=0, keepdims=True) / cnt
    var = jnp.sum(acc_q[...], axis=0, keepdims=True) / cnt - mean * mean
    a = jax.lax.rsqrt(var + EPS_) * gloc[...]
    coef_c[...] = a - 1.0
    coef_b[...] = bloc[...] - mean * a

    def apply_chunk(s, mv):
        xv = xbuf[s]
        t = xv * coef_c[...] + coef_b[...]
        xbuf[s] = xv + t * mv

    for i in range(4):
        c = 28 + i
        apply_chunk(c % RD, mbuf[c % MD])
        write_o(c).start()
        read_m(24 + i, i).start()
        if i == 3:
            write_o(28).wait()
            read_x(4, 4).start()

    for i in range(4):
        c = 24 + i
        read_m(c, i).wait()
        apply_chunk(c % RD, mbuf[i])
        write_o(c).start()
        read_m(i, i).start()
        if i < 3:
            write_o(29 + i).wait()
            read_x(5 + i, 5 + i).start()
        else:
            write_o(24).wait()
            read_x(0, 0).start()

    def p1_body(j2, carry):
        for s in range(RD):
            j = j2 * RD + s

            @pl.when(j < 3)
            def _():
                write_o(25 + j).wait()
                read_x(1 + j, (1 + j) % RD).start()

            @pl.when(j >= 3)
            def _():
                write_o(j - 3).wait()

            @pl.when((j >= 3) & (j + 5 < NCH - RD))
            def _():
                read_x(j + 5, (j + 5) % RD).start()

            read_x(j, s).wait()
            read_m(j, j % MD).wait()
            apply_chunk(s, mbuf[j % MD])
            write_o(j).start()

            @pl.when(j + MD < NCH - RD)
            def _():
                read_m(j + MD, j % MD).start()
        return carry

    jax.lax.fori_loop(0, (NCH - RD) // RD, p1_body, 0)

    for c in range(21, 24):
        write_o(c).wait()


def kernel(x, mask, gamma, beta):
    m = mask.astype(jnp.float32).reshape(ROWS, 1)
    m2 = mask.astype(jnp.float32).reshape(ROWS // 128, 128)
    g = gamma.reshape(1, COLS)
    b = beta.reshape(1, COLS)
    out = pl.pallas_call(
        _bn_kernel,
        in_specs=[
            pl.BlockSpec(memory_space=pl.ANY),
            pl.BlockSpec(memory_space=pl.ANY),
            pl.BlockSpec(memory_space=pl.ANY),
            pl.BlockSpec(memory_space=pl.ANY),
            pl.BlockSpec(memory_space=pl.ANY),
        ],
        out_specs=pl.BlockSpec(memory_space=pl.ANY),
        out_shape=jax.ShapeDtypeStruct((ROWS, COLS), x.dtype),
        scratch_shapes=[
            pltpu.VMEM((RD, CH, COLS), jnp.float32),
            pltpu.VMEM((MD, CH, 1), jnp.float32),
            pltpu.VMEM((1, COLS), jnp.float32),
            pltpu.VMEM((1, COLS), jnp.float32),
            pltpu.VMEM((ROWS // 128, 128), jnp.float32),
            pltpu.VMEM((32, COLS), jnp.float32),
            pltpu.VMEM((32, COLS), jnp.float32),
            pltpu.VMEM((1, COLS), jnp.float32),
            pltpu.VMEM((1, COLS), jnp.float32),
            pltpu.SemaphoreType.DMA((RD,)),
            pltpu.SemaphoreType.DMA((MD,)),
            pltpu.SemaphoreType.DMA((WD,)),
            pltpu.SemaphoreType.DMA((3,)),
        ],
    )(x, m, m2, g, b)
    return out
